# baseline (speedup 1.0000x reference)
import functools

import jax
import jax.numpy as jnp
from jax import lax
from jax.experimental import pallas as pl
from jax.experimental.pallas import tpu as pltpu
from jax.experimental.pallas import tpu_sc as plsc

N = 4 * 4096
D = 1024
NC, NS = 2, 16
NW = NC * NS
B_PER_W = N // NW
CH = 16
NCH = B_PER_W // CH
NBUF = 7

_mesh = plsc.VectorSubcoreMesh(core_axis_name="c", subcore_axis_name="s")


@functools.partial(
    pl.kernel,
    mesh=_mesh,
    out_type=jax.ShapeDtypeStruct((N, D), jnp.float32),
    scratch_types=[
        pltpu.VMEM((B_PER_W,), jnp.int32),
        pltpu.VMEM((NBUF, CH, D), jnp.float32),
        pltpu.SemaphoreType.DMA,
        pltpu.SemaphoreType.DMA,
        pltpu.SemaphoreType.DMA,
        pltpu.SemaphoreType.DMA,
        pltpu.SemaphoreType.DMA,
        pltpu.SemaphoreType.DMA,
        pltpu.SemaphoreType.DMA,
        pltpu.SemaphoreType.DMA,
        pltpu.SemaphoreType.DMA,
        pltpu.SemaphoreType.DMA,
        pltpu.SemaphoreType.DMA,
        pltpu.SemaphoreType.DMA,
        pltpu.SemaphoreType.DMA,
        pltpu.SemaphoreType.DMA,
    ],
)
def _gather_rows(x_hbm, pe_hbm, out_hbm, idx_v, rows_v,
                 g0, g1, g2, g3, g4, g5, g6, s0, s1, s2, s3, s4, s5, s6):
    gsem = (g0, g1, g2, g3, g4, g5, g6)
    ssem = (s0, s1, s2, s3, s4, s5, s6)
    wid = lax.axis_index("s") * NC + lax.axis_index("c")
    base = wid * B_PER_W
    pltpu.sync_copy(x_hbm.at[pl.ds(base, B_PER_W)], idx_v)

    def start_gather(c, b):
        return pltpu.async_copy(
            pe_hbm.at[idx_v.at[pl.ds(c * CH, CH)]], rows_v.at[b], gsem[b])

    gathers = [start_gather(b, b) for b in range(NBUF)]
    stores = [None] * NBUF
    for c in range(NCH):
        b = c % NBUF
        gathers[b].wait()
        stores[b] = pltpu.async_copy(
            rows_v.at[b], out_hbm.at[pl.ds(base + c * CH, CH)], ssem[b])
        gc = c - 1 + NBUF
        if c >= 1 and gc < NCH:
            stores[(c - 1) % NBUF].wait()
            gathers[gc % NBUF] = start_gather(gc, gc % NBUF)
    for i in range(NCH - NBUF, NCH):
        stores[i % NBUF].wait()


def kernel(x, pe):
    out = _gather_rows(x.reshape(N), pe)
    return out.reshape(x.shape + (D,))


# speedup vs baseline: 1.0060x; 1.0060x over previous
import functools

import jax
import jax.numpy as jnp
from jax import lax
from jax.experimental import pallas as pl
from jax.experimental.pallas import tpu as pltpu
from jax.experimental.pallas import tpu_sc as plsc

N = 4 * 4096
D = 1024
NC, NS = 2, 16
NW = NC * NS
B_PER_W = N // NW
CH = 16
NCH = B_PER_W // CH
NBUF = 6

_mesh = plsc.VectorSubcoreMesh(core_axis_name="c", subcore_axis_name="s")


@functools.partial(
    pl.kernel,
    mesh=_mesh,
    out_type=jax.ShapeDtypeStruct((N, D), jnp.float32),
    scratch_types=[
        pltpu.VMEM((B_PER_W,), jnp.int32),
        pltpu.VMEM((NBUF, CH, D), jnp.float32),
        pltpu.SemaphoreType.DMA,
        pltpu.SemaphoreType.DMA,
        pltpu.SemaphoreType.DMA,
        pltpu.SemaphoreType.DMA,
        pltpu.SemaphoreType.DMA,
        pltpu.SemaphoreType.DMA,
        pltpu.SemaphoreType.DMA,
        pltpu.SemaphoreType.DMA,
        pltpu.SemaphoreType.DMA,
        pltpu.SemaphoreType.DMA,
        pltpu.SemaphoreType.DMA,
        pltpu.SemaphoreType.DMA,
    ],
)
def _gather_rows(x_hbm, pe_hbm, out_hbm, idx_v, rows_v,
                 g0, g1, g2, g3, g4, g5, s0, s1, s2, s3, s4, s5):
    gsem = (g0, g1, g2, g3, g4, g5)
    ssem = (s0, s1, s2, s3, s4, s5)
    wid = lax.axis_index("s") * NC + lax.axis_index("c")
    base = wid * B_PER_W
    pltpu.sync_copy(x_hbm.at[pl.ds(base, B_PER_W)], idx_v)

    def start_gather(c, b):
        return pltpu.async_copy(
            pe_hbm.at[idx_v.at[pl.ds(c * CH, CH)]], rows_v.at[b], gsem[b])

    gathers = [start_gather(b, b) for b in range(NBUF)]
    stores = [None] * NBUF
    for c in range(NCH):
        b = c % NBUF
        gathers[b].wait()
        stores[b] = pltpu.async_copy(
            rows_v.at[b], out_hbm.at[pl.ds(base + c * CH, CH)], ssem[b])
        gc = c - 1 + NBUF
        if c >= 1 and gc < NCH:
            stores[(c - 1) % NBUF].wait()
            gathers[gc % NBUF] = start_gather(gc, gc % NBUF)
    for i in range(NCH - NBUF, NCH):
        stores[i % NBUF].wait()


def kernel(x, pe):
    out = _gather_rows(x.reshape(N), pe)
    return out.reshape(x.shape + (D,))
